# Initial kernel scaffold; baseline (speedup 1.0000x reference)
import jax
import jax.numpy as jnp
from jax.experimental import pallas as pl


def kernel(x, embeddings):
    raise NotImplementedError("write your pallas kernel here")


# speedup vs baseline: 2.9773x; 2.9773x over previous
import functools

import jax
import jax.numpy as jnp
from jax import lax
from jax.experimental import pallas as pl
from jax.experimental.pallas import tpu as pltpu
from jax.experimental.pallas import tpu_sc as plsc

D_MODEL = 128
NUM_CORES = 2
NUM_SUBCORES = 16
NUM_WORKERS = NUM_CORES * NUM_SUBCORES
CHUNK = 128


@functools.partial(jax.jit, static_argnames=("b_per_w", "nchunks"))
def _sc_gather(table, idx_grouped, *, b_per_w, nchunks):
    B = NUM_WORKERS * b_per_w
    mesh = plsc.VectorSubcoreMesh(core_axis_name="c", subcore_axis_name="s")

    @functools.partial(
        pl.kernel,
        mesh=mesh,
        out_type=jax.ShapeDtypeStruct((B, D_MODEL), jnp.float32),
        scratch_types=[
            pltpu.VMEM((nchunks, CHUNK), jnp.int32),
            pltpu.VMEM((CHUNK, D_MODEL), jnp.float32),
            pltpu.SemaphoreType.DMA,
        ],
    )
    def k(table_hbm, idx_hbm, out_hbm, idx_v, rows_v, sem):
        wid = lax.axis_index("s") * NUM_CORES + lax.axis_index("c")
        base = wid * b_per_w
        pltpu.sync_copy(idx_hbm.at[wid], idx_v)

        def body(g, _):
            pltpu.async_copy(table_hbm.at[idx_v.at[g]], rows_v, sem).wait()
            pltpu.sync_copy(rows_v, out_hbm.at[pl.ds(base + g * CHUNK, CHUNK)])
            return _

        lax.fori_loop(0, nchunks, body, None)

    return k(table, idx_grouped)


def kernel(x, embeddings):
    orig_shape = x.shape
    idx_flat = x.reshape(-1).astype(jnp.int32)
    B = idx_flat.shape[0]
    assert B % NUM_WORKERS == 0
    b_per_w = B // NUM_WORKERS
    assert b_per_w % CHUNK == 0
    nchunks = b_per_w // CHUNK
    idx_grouped = idx_flat.reshape(NUM_WORKERS, nchunks, CHUNK)
    out = _sc_gather(embeddings, idx_grouped, b_per_w=b_per_w, nchunks=nchunks)
    return out.reshape(*orig_shape, D_MODEL)


# speedup vs baseline: 3.3186x; 1.1146x over previous
import functools

import jax
import jax.numpy as jnp
from jax import lax
from jax.experimental import pallas as pl
from jax.experimental.pallas import tpu as pltpu
from jax.experimental.pallas import tpu_sc as plsc

D_MODEL = 128
NUM_CORES = 2
NUM_SUBCORES = 16
NUM_WORKERS = NUM_CORES * NUM_SUBCORES
CHUNK = 128
NBUF = 5


@functools.partial(jax.jit, static_argnames=("b_per_w", "nchunks"))
def _sc_gather(table, idx_grouped, *, b_per_w, nchunks):
    B = NUM_WORKERS * b_per_w
    mesh = plsc.VectorSubcoreMesh(core_axis_name="c", subcore_axis_name="s")

    @functools.partial(
        pl.kernel,
        mesh=mesh,
        out_type=jax.ShapeDtypeStruct((B, D_MODEL), jnp.float32),
        scratch_types=[
            pltpu.VMEM((nchunks, CHUNK), jnp.int32),
            pltpu.VMEM((NBUF, CHUNK, D_MODEL), jnp.float32),
            pltpu.SemaphoreType.DMA((NBUF,)),
            pltpu.SemaphoreType.DMA((NBUF,)),
        ],
    )
    def k(table_hbm, idx_hbm, out_hbm, idx_v, rows, gsem, wsem):
        wid = lax.axis_index("s") * NUM_CORES + lax.axis_index("c")
        base = wid * b_per_w
        pltpu.sync_copy(idx_hbm.at[wid], idx_v)

        nloops = nchunks // NBUF

        for b in range(NBUF):
            pltpu.async_copy(table_hbm.at[idx_v.at[b]], rows.at[b], gsem.at[b])

        def body(i, _):
            for b in range(NBUF):
                g = i * NBUF + b
                pltpu.make_async_copy(
                    table_hbm.at[pl.ds(0, CHUNK)], rows.at[b], gsem.at[b]
                ).wait()
                pltpu.async_copy(
                    rows.at[b], out_hbm.at[pl.ds(base + g * CHUNK, CHUNK)], wsem.at[b]
                )

            @pl.when(i + 1 < nloops)
            def _refill():
                for b in range(NBUF):
                    g2 = (i + 1) * NBUF + b
                    pltpu.make_async_copy(
                        rows.at[b], out_hbm.at[pl.ds(base, CHUNK)], wsem.at[b]
                    ).wait()
                    pltpu.async_copy(table_hbm.at[idx_v.at[g2]], rows.at[b], gsem.at[b])

            return _

        lax.fori_loop(0, nloops, body, None)

        for b in range(NBUF):
            pltpu.make_async_copy(
                rows.at[b], out_hbm.at[pl.ds(base, CHUNK)], wsem.at[b]
            ).wait()

    return k(table, idx_grouped)


def kernel(x, embeddings):
    orig_shape = x.shape
    idx_flat = x.reshape(-1).astype(jnp.int32)
    B = idx_flat.shape[0]
    assert B % NUM_WORKERS == 0
    b_per_w = B // NUM_WORKERS
    assert b_per_w % CHUNK == 0
    nchunks = b_per_w // CHUNK
    idx_grouped = idx_flat.reshape(NUM_WORKERS, nchunks, CHUNK)
    out = _sc_gather(embeddings, idx_grouped, b_per_w=b_per_w, nchunks=nchunks)
    return out.reshape(*orig_shape, D_MODEL)


# speedup vs baseline: 3.3236x; 1.0015x over previous
import functools

import jax
import jax.numpy as jnp
from jax import lax
from jax.experimental import pallas as pl
from jax.experimental.pallas import tpu as pltpu
from jax.experimental.pallas import tpu_sc as plsc

D_MODEL = 128
NUM_CORES = 2
NUM_SUBCORES = 16
NUM_WORKERS = NUM_CORES * NUM_SUBCORES
CHUNK = 64
NBUF = 10


@functools.partial(jax.jit, static_argnames=("b_per_w", "nchunks"))
def _sc_gather(table, idx_grouped, *, b_per_w, nchunks):
    B = NUM_WORKERS * b_per_w
    mesh = plsc.VectorSubcoreMesh(core_axis_name="c", subcore_axis_name="s")

    @functools.partial(
        pl.kernel,
        mesh=mesh,
        out_type=jax.ShapeDtypeStruct((B, D_MODEL), jnp.float32),
        scratch_types=[
            pltpu.VMEM((nchunks, CHUNK), jnp.int32),
            pltpu.VMEM((NBUF, CHUNK, D_MODEL), jnp.float32),
            pltpu.SemaphoreType.DMA((NBUF,)),
            pltpu.SemaphoreType.DMA((NBUF,)),
        ],
    )
    def k(table_hbm, idx_hbm, out_hbm, idx_v, rows, gsem, wsem):
        wid = lax.axis_index("s") * NUM_CORES + lax.axis_index("c")
        base = wid * b_per_w
        pltpu.sync_copy(idx_hbm.at[wid], idx_v)

        nloops = nchunks // NBUF

        for b in range(NBUF):
            pltpu.async_copy(table_hbm.at[idx_v.at[b]], rows.at[b], gsem.at[b])

        def body(i, _):
            for b in range(NBUF):
                g = i * NBUF + b
                pltpu.make_async_copy(
                    table_hbm.at[pl.ds(0, CHUNK)], rows.at[b], gsem.at[b]
                ).wait()
                pltpu.async_copy(
                    rows.at[b], out_hbm.at[pl.ds(base + g * CHUNK, CHUNK)], wsem.at[b]
                )

            @pl.when(i + 1 < nloops)
            def _refill():
                for b in range(NBUF):
                    g2 = (i + 1) * NBUF + b
                    pltpu.make_async_copy(
                        rows.at[b], out_hbm.at[pl.ds(base, CHUNK)], wsem.at[b]
                    ).wait()
                    pltpu.async_copy(table_hbm.at[idx_v.at[g2]], rows.at[b], gsem.at[b])

            return _

        lax.fori_loop(0, nloops, body, None)

        for b in range(NBUF):
            pltpu.make_async_copy(
                rows.at[b], out_hbm.at[pl.ds(base, CHUNK)], wsem.at[b]
            ).wait()

    return k(table, idx_grouped)


def kernel(x, embeddings):
    orig_shape = x.shape
    idx_flat = x.reshape(-1).astype(jnp.int32)
    B = idx_flat.shape[0]
    assert B % NUM_WORKERS == 0
    b_per_w = B // NUM_WORKERS
    assert b_per_w % CHUNK == 0
    nchunks = b_per_w // CHUNK
    idx_grouped = idx_flat.reshape(NUM_WORKERS, nchunks, CHUNK)
    out = _sc_gather(embeddings, idx_grouped, b_per_w=b_per_w, nchunks=nchunks)
    return out.reshape(*orig_shape, D_MODEL)
